# baseline (speedup 1.0000x reference)
import functools

import jax
import jax.numpy as jnp
from jax.experimental import pallas as pl
from jax.experimental.pallas import tpu as pltpu

_LEVELS = (3072, 6144, 12288, 24576)
_TOPK = (32, 32, 64, 128)
_GAMMA = 1.0
_BETA = 1.0


def _segments(levels, topk):
    starts = (0,) + tuple(levels[:-1])
    return tuple(zip(starts, levels, topk))


def _kth_bits(bits, k):
    r = bits.shape[0]
    lo = jnp.zeros((r, 1), jnp.int32)
    hi = jnp.full((r, 1), 0x7F800000, jnp.int32)

    def body(_, carry):
        lo, hi = carry
        mid = lo + ((hi - lo) >> 1)
        cnt = jnp.sum((bits >= mid).astype(jnp.int32), axis=1, keepdims=True)
        ge = cnt >= k
        return jnp.where(ge, mid, lo), jnp.where(ge, hi, mid)

    lo, hi = jax.lax.fori_loop(0, 31, body, (lo, hi))
    return lo


def _count_ge(sel):
    ones = jnp.where(sel, jnp.bfloat16(1.0), jnp.bfloat16(0.0))
    s = jnp.sum(ones, axis=1, keepdims=True, dtype=jnp.bfloat16)
    return s.astype(jnp.int32)


def _rank_search(arr16, rank, n_iters, hi0):
    r = arr16.shape[0]
    lo = jnp.zeros((r, 1), jnp.int32)
    hi = jnp.full((r, 1), hi0, jnp.int32)
    cnt_hi = jnp.zeros((r, 1), jnp.int32)

    def body(_, carry):
        lo, hi, cnt_hi = carry
        mid = lo + ((hi - lo) >> 1)
        cnt = _count_ge(arr16 >= mid.astype(jnp.bfloat16))
        ge = cnt >= rank
        return (jnp.where(ge, mid, lo), jnp.where(ge, hi, mid),
                jnp.where(ge, cnt_hi, cnt))

    lo, hi, cnt_hi = jax.lax.fori_loop(0, n_iters, body, (lo, hi, cnt_hi))
    return lo, cnt_hi


def _kth_bits3(bits, k):
    r = bits.shape[0]
    hi16 = (bits >> 16).astype(jnp.int16)
    rhi = ((bits >> 8) & 0xFF).astype(jnp.bfloat16)
    rlo = (bits & 0xFF).astype(jnp.bfloat16)

    lo = jnp.zeros((r, 1), jnp.int32)
    hi = jnp.full((r, 1), 0x8000, jnp.int32)
    cnt_hi = jnp.zeros((r, 1), jnp.int32)

    def body_a(_, carry):
        lo, hi, cnt_hi = carry
        mid = lo + ((hi - lo) >> 1)
        cnt = _count_ge(hi16 >= mid.astype(jnp.int16))
        ge = cnt >= k
        return (jnp.where(ge, mid, lo), jnp.where(ge, hi, mid),
                jnp.where(ge, cnt_hi, cnt))

    hstar, _, cnt_gt = jax.lax.fori_loop(0, 15, body_a, (lo, hi, cnt_hi))
    hstar16 = hstar.astype(jnp.int16)
    eq_h = (hi16 >= hstar16) & (hi16 < (hstar + 1).astype(jnp.int16))

    rank_b = k - cnt_gt
    c1 = jnp.where(eq_h, rhi, jnp.bfloat16(-1.0))
    r1, cnt_gt2 = _rank_search(c1, rank_b, 8, 0x100)
    eq_hr = eq_h & (rhi >= r1.astype(jnp.bfloat16)) & (rhi < (r1 + 1).astype(jnp.bfloat16))

    rank_c = k - cnt_gt - cnt_gt2
    c2 = jnp.where(eq_hr, rlo, jnp.bfloat16(-1.0))
    r2, _ = _rank_search(c2, rank_c, 8, 0x100)

    return (hstar << 16) | (r1 << 8) | r2


def _seg_body(k, h_ref, wt_ref, b_ref, *rest):
    out_ref = rest[-1]
    zp = jax.lax.dot_general(
        h_ref[...], wt_ref[...],
        dimension_numbers=(((1,), (0,)), ((), ())),
        preferred_element_type=jnp.float32,
    ) + b_ref[...]
    z = jnp.where(zp > _GAMMA, zp + _BETA, jnp.maximum(zp, 0.0))
    bits = jax.lax.bitcast_convert_type(z, jnp.int32)
    th = _kth_bits3(bits, k)
    out_ref[...] = jax.lax.bitcast_convert_type(
        jnp.where(bits >= th, bits, 0), jnp.float32)


def _seg_call(h_2, w_t, b_2d, prev, d_lat, start, width, k, row_tile):
    n_rows, d_in = h_2.shape
    grid = (n_rows // row_tile,)
    blk = start // width
    in_specs = [
        pl.BlockSpec((row_tile, d_in), lambda i: (i, 0)),
        pl.BlockSpec((d_in, width), lambda i, _b=blk: (0, _b)),
        pl.BlockSpec((1, width), lambda i, _b=blk: (0, _b)),
    ]
    args = [h_2, w_t, b_2d]
    aliases = {}
    if prev is not None:
        in_specs.append(pl.BlockSpec(memory_space=pl.ANY))
        args.append(prev)
        aliases = {3: 0}
    return pl.pallas_call(
        functools.partial(_seg_body, k),
        grid=grid,
        in_specs=in_specs,
        out_specs=pl.BlockSpec((row_tile, width), lambda i, _b=blk: (i, _b)),
        out_shape=jax.ShapeDtypeStruct((n_rows, d_lat), jnp.float32),
        input_output_aliases=aliases,
    )(*args)


def _dual_call(h_2, w_t, b_2d, d_lat, width, k, row_tile):
    n_rows, d_in = h_2.shape
    grid = (2, n_rows // row_tile)
    return pl.pallas_call(
        functools.partial(_seg_body, k),
        grid=grid,
        in_specs=[
            pl.BlockSpec((row_tile, d_in), lambda j, i: (i, 0)),
            pl.BlockSpec((d_in, width), lambda j, i: (0, j)),
            pl.BlockSpec((1, width), lambda j, i: (0, j)),
        ],
        out_specs=pl.BlockSpec((row_tile, width), lambda j, i: (i, j)),
        out_shape=jax.ShapeDtypeStruct((n_rows, d_lat), jnp.float32),
    )(h_2, w_t, b_2d)


def _run(levels, topk, row_tiles, h_2, w_t, b_2d):
    d_lat = levels[-1]
    segs = _segments(levels, topk)
    merge_two = (len(segs) >= 2 and segs[0][2] == segs[1][2]
                 and segs[0][1] - segs[0][0] == segs[1][1] - segs[1][0])
    if merge_two:
        out = _dual_call(h_2, w_t, b_2d, d_lat, segs[0][1] - segs[0][0],
                         segs[0][2], row_tiles[0])
        rest = list(zip(segs[2:], row_tiles[2:]))
    else:
        out = None
        rest = list(zip(segs, row_tiles))
    for (start, end, k), rt in rest:
        out = _seg_call(h_2, w_t, b_2d, out, d_lat, start, end - start, k, rt)
    return out


_ROW_TILES = (512, 512, 128, 64)


def kernel(h_2, W_enc, b_enc):
    w_t = W_enc.T
    b_2d = b_enc.reshape(1, -1)
    return _run(_LEVELS, _TOPK, _ROW_TILES, h_2, w_t, b_2d)


# speedup vs baseline: 1.0021x; 1.0021x over previous
import functools

import jax
import jax.numpy as jnp
from jax.experimental import pallas as pl
from jax.experimental.pallas import tpu as pltpu

_LEVELS = (3072, 6144, 12288, 24576)
_TOPK = (32, 32, 64, 128)
_GAMMA = 1.0
_BETA = 1.0


def _segments(levels, topk):
    starts = (0,) + tuple(levels[:-1])
    return tuple(zip(starts, levels, topk))


def _count_ge(sel):
    ones = jnp.where(sel, jnp.bfloat16(1.0), jnp.bfloat16(0.0))
    s = jnp.sum(ones, axis=1, keepdims=True, dtype=jnp.bfloat16)
    return s.astype(jnp.int32)


def _rank_search(arr16, rank, n_iters, hi0):
    r = arr16.shape[0]
    lo = jnp.zeros((r, 1), jnp.int32)
    hi = jnp.full((r, 1), hi0, jnp.int32)
    cnt_hi = jnp.zeros((r, 1), jnp.int32)

    def body(_, carry):
        lo, hi, cnt_hi = carry
        mid = lo + ((hi - lo) >> 1)
        cnt = _count_ge(arr16 >= mid.astype(jnp.bfloat16))
        ge = cnt >= rank
        return (jnp.where(ge, mid, lo), jnp.where(ge, hi, mid),
                jnp.where(ge, cnt_hi, cnt))

    lo, hi, cnt_hi = jax.lax.fori_loop(0, n_iters, body, (lo, hi, cnt_hi))
    return lo, cnt_hi


def _kth_bits3(bits, k):
    r = bits.shape[0]
    hi16 = (bits >> 16).astype(jnp.int16)
    rhi = ((bits >> 8) & 0xFF).astype(jnp.bfloat16)
    rlo = (bits & 0xFF).astype(jnp.bfloat16)

    lo = jnp.zeros((r, 1), jnp.int32)
    hi = jnp.full((r, 1), 0x8000, jnp.int32)
    cnt_hi = jnp.zeros((r, 1), jnp.int32)

    def body_a(_, carry):
        lo, hi, cnt_hi = carry
        mid = lo + ((hi - lo) >> 1)
        cnt = _count_ge(hi16 >= mid.astype(jnp.int16))
        ge = cnt >= k
        return (jnp.where(ge, mid, lo), jnp.where(ge, hi, mid),
                jnp.where(ge, cnt_hi, cnt))

    hstar, _, cnt_gt = jax.lax.fori_loop(0, 15, body_a, (lo, hi, cnt_hi))
    hstar16 = hstar.astype(jnp.int16)
    eq_h = (hi16 >= hstar16) & (hi16 < (hstar + 1).astype(jnp.int16))

    rank_b = k - cnt_gt
    c1 = jnp.where(eq_h, rhi, jnp.bfloat16(-1.0))
    r1, cnt_gt2 = _rank_search(c1, rank_b, 8, 0x100)
    eq_hr = eq_h & (rhi >= r1.astype(jnp.bfloat16)) & (rhi < (r1 + 1).astype(jnp.bfloat16))

    rank_c = k - cnt_gt - cnt_gt2
    c2 = jnp.where(eq_hr, rlo, jnp.bfloat16(-1.0))
    r2, _ = _rank_search(c2, rank_c, 8, 0x100)

    return (hstar << 16) | (r1 << 8) | r2


def _seg_body(k, h_ref, wt_ref, b_ref, *rest):
    out_ref = rest[-1]
    zp = jax.lax.dot_general(
        h_ref[...], wt_ref[...],
        dimension_numbers=(((1,), (0,)), ((), ())),
        preferred_element_type=jnp.float32,
    ) + b_ref[...]
    z = jnp.where(zp > _GAMMA, zp + _BETA, jnp.maximum(zp, 0.0))
    bits = jax.lax.bitcast_convert_type(z, jnp.int32)
    th = _kth_bits3(bits, k)
    out_ref[...] = jnp.where(bits >= th, z, 0.0)


def _seg_call(h_2, w_t, b_2d, prev, d_lat, start, width, k, row_tile):
    n_rows, d_in = h_2.shape
    grid = (n_rows // row_tile,)
    blk = start // width
    in_specs = [
        pl.BlockSpec((row_tile, d_in), lambda i: (i, 0)),
        pl.BlockSpec((d_in, width), lambda i, _b=blk: (0, _b)),
        pl.BlockSpec((1, width), lambda i, _b=blk: (0, _b)),
    ]
    args = [h_2, w_t, b_2d]
    aliases = {}
    if prev is not None:
        in_specs.append(pl.BlockSpec(memory_space=pl.ANY))
        args.append(prev)
        aliases = {3: 0}
    return pl.pallas_call(
        functools.partial(_seg_body, k),
        grid=grid,
        in_specs=in_specs,
        out_specs=pl.BlockSpec((row_tile, width), lambda i, _b=blk: (i, _b)),
        out_shape=jax.ShapeDtypeStruct((n_rows, d_lat), jnp.float32),
        input_output_aliases=aliases,
    )(*args)


def _dual_call(h_2, w_t, b_2d, d_lat, width, k, row_tile):
    n_rows, d_in = h_2.shape
    grid = (2, n_rows // row_tile)
    return pl.pallas_call(
        functools.partial(_seg_body, k),
        grid=grid,
        in_specs=[
            pl.BlockSpec((row_tile, d_in), lambda j, i: (i, 0)),
            pl.BlockSpec((d_in, width), lambda j, i: (0, j)),
            pl.BlockSpec((1, width), lambda j, i: (0, j)),
        ],
        out_specs=pl.BlockSpec((row_tile, width), lambda j, i: (i, j)),
        out_shape=jax.ShapeDtypeStruct((n_rows, d_lat), jnp.float32),
    )(h_2, w_t, b_2d)


def _run(levels, topk, row_tiles, h_2, w_t, b_2d):
    d_lat = levels[-1]
    segs = _segments(levels, topk)
    merge_two = (len(segs) >= 2 and segs[0][2] == segs[1][2]
                 and segs[0][1] - segs[0][0] == segs[1][1] - segs[1][0])
    if merge_two:
        out = _dual_call(h_2, w_t, b_2d, d_lat, segs[0][1] - segs[0][0],
                         segs[0][2], row_tiles[0])
        rest = list(zip(segs[2:], row_tiles[2:]))
    else:
        out = None
        rest = list(zip(segs, row_tiles))
    for (start, end, k), rt in rest:
        out = _seg_call(h_2, w_t, b_2d, out, d_lat, start, end - start, k, rt)
    return out


_ROW_TILES = (512, 512, 128, 64)


def kernel(h_2, W_enc, b_enc):
    w_t = W_enc.T
    b_2d = b_enc.reshape(1, -1)
    return _run(_LEVELS, _TOPK, _ROW_TILES, h_2, w_t, b_2d)
